# baseline (speedup 1.0000x reference)
import functools

import jax
import jax.numpy as jnp
from jax import lax
from jax.experimental import pallas as pl
from jax.experimental.pallas import tpu as pltpu
from jax.experimental.pallas import tpu_sc as plsc

B = 16384
L = 5
EMB_DIM = 64
CLASS_DIM = 16
FEAT = 80
FPAD = 128
ND = 175
NUM_CORES = 2
NUM_SUBCORES = 16
NW = NUM_CORES * NUM_SUBCORES
IPW = B // NW
CH = 128
NCH = IPW // CH
NSTEP = 2 * NCH
INV_L = 0.2


def _sc_pool_body(emb_hbm, ctab_hbm, ida_hbm, cida_hbm, idb_hbm, cidb_hbm,
                  outa_hbm, outb_hbm,
                  idxe_a, idxc_a, idxe_b, idxc_b,
                  ctab, bufe0, bufe1, oute,
                  sem0, sem1, semw, *, hbase, ipw, nch):
  wid = lax.axis_index("s") * NUM_CORES + lax.axis_index("c")
  base = wid * ipw
  nstep = 2 * nch

  pltpu.sync_copy(ctab_hbm, ctab)

  for src, dst in ((ida_hbm, idxe_a), (cida_hbm, idxc_a),
                   (idb_hbm, idxe_b), (cidb_hbm, idxc_b)):
    for k in range(L):
      pltpu.sync_copy(src.at[pl.ds(k * B + hbase + base, ipw)], dst.at[k])

  zero = jnp.zeros((16,), jnp.float32)

  def zero_bufs(c, carry, be=None):
    for j in range(EMB_DIM // 16):
      be[c, pl.ds(16 * j, 16)] = zero
    return carry

  for be in (bufe0, bufe1):
    lax.fori_loop(0, CH, functools.partial(zero_bufs, be=be), 0,
                  unroll=False)

  steps = ([(idxe_a, idxc_a, outa_hbm, cc) for cc in range(nch)]
           + [(idxe_b, idxc_b, outb_hbm, cc) for cc in range(nch)])
  bufs = [(bufe0, sem0), (bufe1, sem1)]

  def fire(s):
    idx, _, _, cc = steps[s]
    be, sem = bufs[s % 2]
    cps = []
    for k in range(L):
      cps.append(pltpu.async_copy(
          emb_hbm.at[idx.at[k, pl.ds(cc * CH, CH)]], be, sem, add=True))
    return cps

  pend = fire(0)
  wpend = []
  for s in range(nstep):
    nxt = fire(s + 1) if s + 1 < nstep else []
    for cp in pend:
      cp.wait()
    pend = nxt
    for cp in wpend:
      cp.wait()
    be, _ = bufs[s % 2]
    _, cidx, out_hbm, cc = steps[s]

    def red_body(c, carry, be=be, cidx=cidx, cc=cc):
      for j in range(EMB_DIM // 16):
        oute[c, pl.ds(16 * j, 16)] = be[c, pl.ds(16 * j, 16)]
        be[c, pl.ds(16 * j, 16)] = zero
      accc = None
      for k in range(L):
        cid = cidx[k, pl.ds(cc * CH + c, 1)][0]
        row = ctab[cid, :]
        accc = row if accc is None else accc + row
      oute[c, pl.ds(EMB_DIM, CLASS_DIM)] = accc
      return carry

    lax.fori_loop(0, CH, red_body, 0, unroll=False)
    wpend = [
        pltpu.async_copy(
            oute, out_hbm.at[pl.ds(base + cc * CH, CH), pl.ds(0, FEAT)],
            semw),
    ]
  for cp in wpend:
    cp.wait()


def _sc_pool(emb_table, class_table, ida, cida, idb, cidb, half, nhalves):
  hb = B // nhalves
  ipw = hb // NW
  nch = ipw // CH
  mesh = plsc.VectorSubcoreMesh(core_axis_name="c", subcore_axis_name="s")
  out_type = (jax.ShapeDtypeStruct((hb, FPAD), jnp.float32),
              jax.ShapeDtypeStruct((hb, FPAD), jnp.float32))
  scratch = [
      pltpu.VMEM((L, ipw), jnp.int32),
      pltpu.VMEM((L, ipw), jnp.int32),
      pltpu.VMEM((L, ipw), jnp.int32),
      pltpu.VMEM((L, ipw), jnp.int32),
      pltpu.VMEM((1000, CLASS_DIM), jnp.float32),
      pltpu.VMEM((CH, EMB_DIM), jnp.float32),
      pltpu.VMEM((CH, EMB_DIM), jnp.float32),
      pltpu.VMEM((CH, FEAT), jnp.float32),
      pltpu.SemaphoreType.DMA,
      pltpu.SemaphoreType.DMA,
      pltpu.SemaphoreType.DMA,
  ]
  body = functools.partial(_sc_pool_body, hbase=half * hb, ipw=ipw,
                           nch=nch)
  fn = pl.kernel(body, out_type=out_type, mesh=mesh,
                 scratch_types=scratch,
                 compiler_params=pltpu.CompilerParams(
                     use_tc_tiling_on_sc=False))
  return fn(emb_table, class_table, ida, cida, idb, cidb)


BM = 1024
BF = jnp.bfloat16


def _tc_mlp_body(fa, fb, nd_a, nd_b,
                 w1fa, w1fb, wnd_a, wnd_b, b1, w2, b2, w3, b3, out):
  f32 = jnp.float32
  xa = fa[:, pl.ds(0, FEAT)].astype(BF)
  xb = fb[:, pl.ds(0, FEAT)].astype(BF)
  na = nd_a[...].astype(BF)
  nb = nd_b[...].astype(BF)
  h = jnp.dot(xa, w1fa[...], preferred_element_type=f32)
  h = h + jnp.dot(na, wnd_a[...], preferred_element_type=f32)
  h = h + jnp.dot(xb, w1fb[...], preferred_element_type=f32)
  h = h + jnp.dot(nb, wnd_b[...], preferred_element_type=f32)
  h = jnp.maximum(h + b1[...], 0.0).astype(BF)
  h2 = jnp.dot(h, w2[...], preferred_element_type=f32)
  h2 = jnp.maximum(h2 + b2[...], 0.0).astype(BF)
  o = jnp.dot(h2, w3[...], preferred_element_type=f32)
  out[...] = jax.nn.sigmoid(o + b3[...])


def _tc_mlp(fa, fb, nd_a, nd_b, w1fa, w1fb, wnd_a, wnd_b, b1, w2, b2, w3,
            b3):
  grid = (fa.shape[0] // BM,)
  row = lambda i: (i, 0)
  const = lambda i: (0, 0)
  in_specs = [
      pl.BlockSpec((BM, FPAD), row),
      pl.BlockSpec((BM, FPAD), row),
      pl.BlockSpec((BM, ND), row),
      pl.BlockSpec((BM, ND), row),
      pl.BlockSpec((FEAT, 512), const),
      pl.BlockSpec((FEAT, 512), const),
      pl.BlockSpec((ND, 512), const),
      pl.BlockSpec((ND, 512), const),
      pl.BlockSpec((1, 512), const),
      pl.BlockSpec((512, 256), const),
      pl.BlockSpec((1, 256), const),
      pl.BlockSpec((256, 1), const),
      pl.BlockSpec((1, 1), const),
  ]
  out = pl.pallas_call(
      _tc_mlp_body,
      grid=grid,
      in_specs=in_specs,
      out_specs=pl.BlockSpec((BM, 1), row),
      out_shape=jax.ShapeDtypeStruct((fa.shape[0], 1), jnp.float32),
      compiler_params=pltpu.CompilerParams(
          dimension_semantics=("parallel",)),
  )(fa, fb, nd_a, nd_b, w1fa, w1fb, wnd_a, wnd_b, b1, w2, b2, w3, b3)
  return out


def kernel(team_a_ids, team_b_ids, team_a_numerical, team_b_numerical,
           team_a_class_ids, team_b_class_ids, team_a_damage_one_hot,
           team_b_damage_one_hot, emb_table, class_table, W1, b1, W2, b2,
           W3, b3):
  tflat = lambda ids: ids.astype(jnp.int32).T.reshape(L * B)
  ida, cida = tflat(team_a_ids), tflat(team_a_class_ids)
  idb, cidb = tflat(team_b_ids), tflat(team_b_class_ids)

  NH = 2
  halves = [_sc_pool(emb_table, class_table, ida, cida, idb, cidb, h, NH)
            for h in range(NH)]

  nd_a = jnp.concatenate([team_a_numerical.reshape(B, 160),
                          team_a_damage_one_hot.reshape(B, 15)], axis=1)
  nd_b = jnp.concatenate([team_b_numerical.reshape(B, 160),
                          team_b_damage_one_hot.reshape(B, 15)], axis=1)

  w1fa = (jnp.concatenate([W1[0:64], W1[96:112]], 0) * INV_L).astype(BF)
  w1fb = (jnp.concatenate([W1[115:179], W1[211:227]], 0) * INV_L).astype(BF)
  wnd_a = (jnp.concatenate([jnp.tile(W1[64:96], (5, 1)),
                            jnp.tile(W1[112:115], (5, 1))], 0)
           * INV_L).astype(BF)
  wnd_b = (jnp.concatenate([jnp.tile(W1[179:211], (5, 1)),
                            jnp.tile(W1[227:230], (5, 1))], 0)
           * INV_L).astype(BF)

  hb = B // NH
  outs = []
  for h, (fa, fb) in enumerate(halves):
    sl = slice(h * hb, (h + 1) * hb)
    outs.append(_tc_mlp(fa, fb, nd_a[sl], nd_b[sl], w1fa, w1fb, wnd_a,
                        wnd_b, b1.reshape(1, 512), W2.astype(BF),
                        b2.reshape(1, 256), W3.astype(BF),
                        b3.reshape(1, 1)))
  return jnp.concatenate(outs, axis=0).reshape(B)


# speedup vs baseline: 1.0020x; 1.0020x over previous
import functools

import jax
import jax.numpy as jnp
from jax import lax
from jax.experimental import pallas as pl
from jax.experimental.pallas import tpu as pltpu
from jax.experimental.pallas import tpu_sc as plsc

B = 16384
L = 5
EMB_DIM = 64
CLASS_DIM = 16
FEAT = 80
FPAD = 128
ND = 175
NUM_CORES = 2
NUM_SUBCORES = 16
NW = NUM_CORES * NUM_SUBCORES
IPW = B // NW
CH = 128
NCH = IPW // CH
NSTEP = 2 * NCH
INV_L = 0.2


def _sc_pool_body(emb_hbm, ctab_hbm, ida_hbm, cida_hbm, idb_hbm, cidb_hbm,
                  outa_hbm, outb_hbm,
                  idxe_a, idxc_a, idxe_b, idxc_b,
                  ctab, bufe0, bufe1, oute,
                  sem0, sem1, semw, *, hbase, ipw, nch):
  wid = lax.axis_index("s") * NUM_CORES + lax.axis_index("c")
  base = wid * ipw
  nstep = 2 * nch

  pltpu.sync_copy(ctab_hbm, ctab)

  for src, dst in ((ida_hbm, idxe_a), (cida_hbm, idxc_a),
                   (idb_hbm, idxe_b), (cidb_hbm, idxc_b)):
    for k in range(L):
      pltpu.sync_copy(src.at[pl.ds(k * B + hbase + base, ipw)], dst.at[k])

  zero = jnp.zeros((16,), jnp.float32)

  def zero_bufs(c, carry, be=None):
    for j in range(EMB_DIM // 16):
      be[c, pl.ds(16 * j, 16)] = zero
    return carry

  for be in (bufe0, bufe1):
    lax.fori_loop(0, CH, functools.partial(zero_bufs, be=be), 0,
                  unroll=False)

  steps = ([(idxe_a, idxc_a, outa_hbm, cc) for cc in range(nch)]
           + [(idxe_b, idxc_b, outb_hbm, cc) for cc in range(nch)])
  bufs = [(bufe0, sem0), (bufe1, sem1)]

  def fire(s):
    idx, _, _, cc = steps[s]
    be, sem = bufs[s % 2]
    cps = []
    for k in range(L):
      cps.append(pltpu.async_copy(
          emb_hbm.at[idx.at[k, pl.ds(cc * CH, CH)]], be, sem, add=True))
    return cps

  pend = fire(0)
  wpend = []
  for s in range(nstep):
    nxt = fire(s + 1) if s + 1 < nstep else []
    for cp in pend:
      cp.wait()
    pend = nxt
    for cp in wpend:
      cp.wait()
    be, _ = bufs[s % 2]
    _, cidx, out_hbm, cc = steps[s]

    def red_body(c, carry, be=be, cidx=cidx, cc=cc):
      for j in range(EMB_DIM // 16):
        oute[c, pl.ds(16 * j, 16)] = be[c, pl.ds(16 * j, 16)]
        be[c, pl.ds(16 * j, 16)] = zero
      accc = None
      for k in range(L):
        cid = cidx[k, pl.ds(cc * CH + c, 1)][0]
        row = ctab[cid, :]
        accc = row if accc is None else accc + row
      oute[c, pl.ds(EMB_DIM, CLASS_DIM)] = accc
      return carry

    lax.fori_loop(0, CH, red_body, 0, unroll=False)
    wpend = [
        pltpu.async_copy(
            oute, out_hbm.at[pl.ds(base + cc * CH, CH), pl.ds(0, FEAT)],
            semw),
    ]
  for cp in wpend:
    cp.wait()


def _sc_pool(emb_table, class_table, ida, cida, idb, cidb, half, nhalves):
  hb = B // nhalves
  ipw = hb // NW
  nch = ipw // CH
  mesh = plsc.VectorSubcoreMesh(core_axis_name="c", subcore_axis_name="s")
  out_type = (jax.ShapeDtypeStruct((hb, FPAD), jnp.float32),
              jax.ShapeDtypeStruct((hb, FPAD), jnp.float32))
  scratch = [
      pltpu.VMEM((L, ipw), jnp.int32),
      pltpu.VMEM((L, ipw), jnp.int32),
      pltpu.VMEM((L, ipw), jnp.int32),
      pltpu.VMEM((L, ipw), jnp.int32),
      pltpu.VMEM((1000, CLASS_DIM), jnp.float32),
      pltpu.VMEM((CH, EMB_DIM), jnp.float32),
      pltpu.VMEM((CH, EMB_DIM), jnp.float32),
      pltpu.VMEM((CH, FEAT), jnp.float32),
      pltpu.SemaphoreType.DMA,
      pltpu.SemaphoreType.DMA,
      pltpu.SemaphoreType.DMA,
  ]
  body = functools.partial(_sc_pool_body, hbase=half * hb, ipw=ipw,
                           nch=nch)
  fn = pl.kernel(body, out_type=out_type, mesh=mesh,
                 scratch_types=scratch,
                 compiler_params=pltpu.CompilerParams(
                     use_tc_tiling_on_sc=False))
  return fn(emb_table, class_table, ida, cida, idb, cidb)


BM = 1024
BF = jnp.bfloat16


def _tc_mlp_body(fa, fb, nd_a, nd_b,
                 w1fa, w1fb, wnd_a, wnd_b, b1, w2, b2, w3, b3, out):
  f32 = jnp.float32
  xa = fa[:, pl.ds(0, FEAT)].astype(BF)
  xb = fb[:, pl.ds(0, FEAT)].astype(BF)
  na = nd_a[...].astype(BF)
  nb = nd_b[...].astype(BF)
  h = jnp.dot(xa, w1fa[...], preferred_element_type=f32)
  h = h + jnp.dot(na, wnd_a[...], preferred_element_type=f32)
  h = h + jnp.dot(xb, w1fb[...], preferred_element_type=f32)
  h = h + jnp.dot(nb, wnd_b[...], preferred_element_type=f32)
  h = jnp.maximum(h + b1[...], 0.0).astype(BF)
  h2 = jnp.dot(h, w2[...], preferred_element_type=f32)
  h2 = jnp.maximum(h2 + b2[...], 0.0).astype(BF)
  o = jnp.dot(h2, w3[...], preferred_element_type=f32)
  out[...] = jax.nn.sigmoid(o + b3[...])


def _tc_mlp(fa, fb, nd_a, nd_b, w1fa, w1fb, wnd_a, wnd_b, b1, w2, b2, w3,
            b3):
  grid = (fa.shape[0] // BM,)
  row = lambda i: (i, 0)
  const = lambda i: (0, 0)
  in_specs = [
      pl.BlockSpec((BM, FPAD), row),
      pl.BlockSpec((BM, FPAD), row),
      pl.BlockSpec((BM, ND), row),
      pl.BlockSpec((BM, ND), row),
      pl.BlockSpec((FEAT, 512), const),
      pl.BlockSpec((FEAT, 512), const),
      pl.BlockSpec((ND, 512), const),
      pl.BlockSpec((ND, 512), const),
      pl.BlockSpec((1, 512), const),
      pl.BlockSpec((512, 256), const),
      pl.BlockSpec((1, 256), const),
      pl.BlockSpec((256, 1), const),
      pl.BlockSpec((1, 1), const),
  ]
  out = pl.pallas_call(
      _tc_mlp_body,
      grid=grid,
      in_specs=in_specs,
      out_specs=pl.BlockSpec((BM, 1), row),
      out_shape=jax.ShapeDtypeStruct((fa.shape[0], 1), jnp.float32),
      compiler_params=pltpu.CompilerParams(
          dimension_semantics=("parallel",)),
  )(fa, fb, nd_a, nd_b, w1fa, w1fb, wnd_a, wnd_b, b1, w2, b2, w3, b3)
  return out


def kernel(team_a_ids, team_b_ids, team_a_numerical, team_b_numerical,
           team_a_class_ids, team_b_class_ids, team_a_damage_one_hot,
           team_b_damage_one_hot, emb_table, class_table, W1, b1, W2, b2,
           W3, b3):
  tflat = lambda ids: ids.astype(jnp.int32).T.reshape(L * B)
  ida, cida = tflat(team_a_ids), tflat(team_a_class_ids)
  idb, cidb = tflat(team_b_ids), tflat(team_b_class_ids)

  NH = 1
  halves = [_sc_pool(emb_table, class_table, ida, cida, idb, cidb, h, NH)
            for h in range(NH)]

  nd_a = jnp.concatenate([team_a_numerical.reshape(B, 160),
                          team_a_damage_one_hot.reshape(B, 15)], axis=1)
  nd_b = jnp.concatenate([team_b_numerical.reshape(B, 160),
                          team_b_damage_one_hot.reshape(B, 15)], axis=1)

  w1fa = (jnp.concatenate([W1[0:64], W1[96:112]], 0) * INV_L).astype(BF)
  w1fb = (jnp.concatenate([W1[115:179], W1[211:227]], 0) * INV_L).astype(BF)
  wnd_a = (jnp.concatenate([jnp.tile(W1[64:96], (5, 1)),
                            jnp.tile(W1[112:115], (5, 1))], 0)
           * INV_L).astype(BF)
  wnd_b = (jnp.concatenate([jnp.tile(W1[179:211], (5, 1)),
                            jnp.tile(W1[227:230], (5, 1))], 0)
           * INV_L).astype(BF)

  hb = B // NH
  outs = []
  for h, (fa, fb) in enumerate(halves):
    sl = slice(h * hb, (h + 1) * hb)
    outs.append(_tc_mlp(fa, fb, nd_a[sl], nd_b[sl], w1fa, w1fb, wnd_a,
                        wnd_b, b1.reshape(1, 512), W2.astype(BF),
                        b2.reshape(1, 256), W3.astype(BF),
                        b3.reshape(1, 1)))
  return jnp.concatenate(outs, axis=0).reshape(B)
